# baseline (speedup 1.0000x reference)
import functools

import jax
import jax.numpy as jnp
import numpy as np
from jax import lax
from jax.experimental import pallas as pl
from jax.experimental.pallas import tpu as pltpu
from jax.experimental.pallas import tpu_sc as plsc

BATCH = 16384
VOCAB = 100000
NNZ = 327680
D = 32
DU = 128
NOUT = D + DU

NC = 2
NS = 16
NW = NC * NS
L = 16

CHUNK = 1024
SEG = 128
NSEG = CHUNK // SEG
EPW = NNZ // NW
NCHUNK = EPW // CHUNK
SEGPW = EPW // SEG
RPS = BATCH // NS

RB = BATCH // NW
FB = 256
FBP = 264


_WBM = 1024


def _wprep_body(wt_ref, o_ref):
    row = lax.broadcasted_iota(jnp.int32, (D, D), 0)
    col = lax.broadcasted_iota(jnp.int32, (D, D), 1)
    eye = (row == col).astype(jnp.float32)
    t = lax.dot_general(wt_ref[...], eye, (((0,), (0,)), ((), ())),
                        preferred_element_type=jnp.float32)
    t3 = t.reshape(_WBM, 4, D)
    row8 = lax.broadcasted_iota(jnp.int32, (D, 128), 0)
    col8 = lax.broadcasted_iota(jnp.int32, (D, 128), 1)
    acc = jnp.zeros((_WBM, 128), jnp.float32)
    for j in range(4):
        ej = (col8 == row8 + 32 * j).astype(jnp.float32)
        acc = acc + lax.dot_general(
            t3[:, j, :], ej, (((1,), (0,)), ((), ())),
            preferred_element_type=jnp.float32)
    o_ref[...] = acc


def _w_prep(w_t):
    out = pl.pallas_call(
        _wprep_body,
        grid=(pl.cdiv(VOCAB, 4 * _WBM),),
        in_specs=[pl.BlockSpec((D, 4 * _WBM), lambda i: (0, i))],
        out_specs=pl.BlockSpec((_WBM, 128), lambda i: (i, 0)),
        out_shape=jax.ShapeDtypeStruct((VOCAB * D // 128, 128), jnp.float32),
    )(w_t)
    return out.reshape(VOCAB, D)


def _sc_body(rows_hbm, cols_hbm, vals_hbm, w_hbm, out_hbm,
             rows_all, cols_all, vals_all, gbuf0, gbuf1, acc,
             sem_i, sem_g0, sem_g1, sem_s0, sem_s1):
    c = lax.axis_index("c")
    s = lax.axis_index("s")
    wid = c * NS + s
    seg0 = wid * SEGPW

    idx_descs = [
        pltpu.async_copy(rows_hbm.at[pl.ds(seg0, SEGPW)], rows_all, sem_i),
        pltpu.async_copy(cols_hbm.at[pl.ds(seg0, SEGPW)], cols_all, sem_i),
        pltpu.async_copy(vals_hbm.at[pl.ds(seg0, SEGPW)], vals_all, sem_i),
    ]

    zero = jnp.zeros((L,), jnp.float32)

    def zero_body(i, carry):
        gbuf0[i, pl.ds(0, L)] = zero
        gbuf0[i, pl.ds(L, L)] = zero
        return carry

    lax.fori_loop(0, CHUNK, zero_body, 0)
    pltpu.sync_copy(gbuf0, acc.at[pl.ds(s * RPS, RPS)])
    plsc.subcore_barrier()
    for dsc in idx_descs:
        dsc.wait()

    gbufs = (gbuf0, gbuf1)
    sems_g = (sem_g0, sem_g1)
    sems_s = (sem_s0, sem_s1)

    def fire_gather(k):
        gb, sem = gbufs[k % 2], sems_g[k % 2]
        return [
            pltpu.async_copy(w_hbm.at[cols_all.at[k * NSEG + j]],
                             gb.at[pl.ds(j * SEG, SEG)], sem)
            for j in range(NSEG)
        ]

    lane_ids = [jnp.full((L, 1), i, jnp.int32) for i in range(L)]
    gather_dn = lax.GatherDimensionNumbers(
        offset_dims=(), collapsed_slice_dims=(0,), start_index_map=(0,))

    descs_g = {0: fire_gather(0)}
    descs_s = {}
    for k in range(NCHUNK):
        gb = gbufs[k % 2]
        for dsc in descs_g.pop(k):
            dsc.wait()
        if k + 1 < NCHUNK:
            if k >= 1:
                for dsc in descs_s.pop(k - 1):
                    dsc.wait()
            descs_g[k + 1] = fire_gather(k + 1)

        def run_scale(gb, k):
            @plsc.parallel_loop(0, CHUNK // L, unroll=4)
            def scale_body(g):
                v = vals_all[k * NSEG + g // 8, pl.ds((g % 8) * L, L)]
                for i in range(L):
                    e = g * L + i
                    val = lax.gather(
                        v, lane_ids[i], gather_dn, slice_sizes=(1,),
                        mode=lax.GatherScatterMode.PROMISE_IN_BOUNDS)
                    gb[e, pl.ds(0, L)] = gb[e, pl.ds(0, L)] * val
                    gb[e, pl.ds(L, L)] = gb[e, pl.ds(L, L)] * val

        run_scale(gb, k)

        descs_s[k] = [
            pltpu.async_copy(gb.at[pl.ds(j * SEG, SEG)],
                             acc.at[rows_all.at[k * NSEG + j]],
                             sems_s[k % 2], add=True)
            for j in range(NSEG)
        ]
    for k in (NCHUNK - 2, NCHUNK - 1):
        for dsc in descs_s.pop(k):
            dsc.wait()
    plsc.subcore_barrier()

    pltpu.sync_copy(acc.at[pl.ds(s * RPS, RPS)],
                    out_hbm.at[c, pl.ds(s * RPS, RPS)])


_sc_accumulate = pl.kernel(
    _sc_body,
    out_type=jax.ShapeDtypeStruct((NC, BATCH, D), jnp.float32),
    mesh=plsc.VectorSubcoreMesh(core_axis_name="c", subcore_axis_name="s"),
    compiler_params=pltpu.CompilerParams(
        use_tc_tiling_on_sc=False, needs_layout_passes=False),
    scratch_types=[
        pltpu.VMEM((SEGPW, SEG), jnp.int32),
        pltpu.VMEM((SEGPW, SEG), jnp.int32),
        pltpu.VMEM((SEGPW, SEG), jnp.float32),
        pltpu.VMEM((CHUNK, D), jnp.float32),
        pltpu.VMEM((CHUNK, D), jnp.float32),
        pltpu.VMEM_SHARED((BATCH, D), jnp.float32),
        pltpu.SemaphoreType.DMA,
        pltpu.SemaphoreType.DMA,
        pltpu.SemaphoreType.DMA,
        pltpu.SemaphoreType.DMA,
        pltpu.SemaphoreType.DMA,
    ],
)


def _fin_body(p_hbm, d_hbm, b_hbm, out_hbm, p0b, p1b, dbuf, obuf, bbuf,
              sem_i, sem_o):
    c = lax.axis_index("c")
    s = lax.axis_index("s")
    wid = c * NS + s
    pltpu.sync_copy(b_hbm, bbuf)
    b0 = bbuf[pl.ds(0, L)]
    b1 = bbuf[pl.ds(L, L)]
    iota = lax.iota(jnp.int32, L)

    for it in range(RB // FB):
        r0 = wid * RB + it * FB
        in_descs = [
            pltpu.async_copy(p_hbm.at[0, pl.ds(r0, FB)], p0b, sem_i),
            pltpu.async_copy(p_hbm.at[1, pl.ds(r0, FB)], p1b, sem_i),
            pltpu.async_copy(d_hbm.at[pl.ds(r0, FB)], dbuf, sem_i),
        ]
        for dsc in in_descs:
            dsc.wait()

        @plsc.parallel_loop(0, FB, unroll=4)
        def ent_body(r):
            rv = jnp.full((L,), r, jnp.int32)
            sp0 = p0b[r, pl.ds(0, L)] + p1b[r, pl.ds(0, L)] + b0
            sp1 = p0b[r, pl.ds(L, L)] + p1b[r, pl.ds(L, L)] + b1
            plsc.store_scatter(obuf, [iota, rv], sp0)
            plsc.store_scatter(obuf, [iota + L, rv], sp1)
            for h in range(DU // L):
                dv = dbuf[r, pl.ds(h * L, L)]
                plsc.store_scatter(obuf, [iota + (D + h * L), rv], dv)

        pltpu.async_copy(obuf.at[:, pl.ds(0, FB)],
                         out_hbm.at[:, pl.ds(r0, FB)], sem_o).wait()


_sc_finish = pl.kernel(
    _fin_body,
    out_type=jax.ShapeDtypeStruct((NOUT, BATCH), jnp.float32),
    mesh=plsc.VectorSubcoreMesh(core_axis_name="c", subcore_axis_name="s"),
    compiler_params=pltpu.CompilerParams(
        use_tc_tiling_on_sc=False, needs_layout_passes=False),
    scratch_types=[
        pltpu.VMEM((FB, D), jnp.float32),
        pltpu.VMEM((FB, D), jnp.float32),
        pltpu.VMEM((FB, DU), jnp.float32),
        pltpu.VMEM((NOUT, FBP), jnp.float32),
        pltpu.VMEM((D,), jnp.float32),
        pltpu.SemaphoreType.DMA,
        pltpu.SemaphoreType.DMA,
    ],
)


def kernel(sparse_rows, sparse_cols, sparse_vals, dense_feat, W, b):
    rows2d = sparse_rows.astype(jnp.int32).reshape(NNZ // SEG, SEG)
    cols2d = sparse_cols.astype(jnp.int32).reshape(NNZ // SEG, SEG)
    vals2d = sparse_vals.reshape(NNZ // SEG, SEG)
    w_sc = _w_prep(W.T)
    partial = _sc_accumulate(rows2d, cols2d, vals2d, w_sc)
    out_t = _sc_finish(partial, dense_feat, b)
    return out_t.T


# speedup vs baseline: 1.0085x; 1.0085x over previous
import functools

import jax
import jax.numpy as jnp
import numpy as np
from jax import lax
from jax.experimental import pallas as pl
from jax.experimental.pallas import tpu as pltpu
from jax.experimental.pallas import tpu_sc as plsc

BATCH = 16384
VOCAB = 100000
NNZ = 327680
D = 32
DU = 128
NOUT = D + DU

NC = 2
NS = 16
NW = NC * NS
L = 16

CHUNK = 1024
SEG = 128
NSEG = CHUNK // SEG
EPW = NNZ // NW
NCHUNK = EPW // CHUNK
SEGPW = EPW // SEG
RPS = BATCH // NS

RB = BATCH // NW
FB = 256
FBP = 264


_WBM = 1024


def _wprep_body(wt_ref, o_ref):
    row = lax.broadcasted_iota(jnp.int32, (D, D), 0)
    col = lax.broadcasted_iota(jnp.int32, (D, D), 1)
    eye = (row == col).astype(jnp.float32)
    t = lax.dot_general(wt_ref[...], eye, (((0,), (0,)), ((), ())),
                        preferred_element_type=jnp.float32)
    t3 = t.reshape(_WBM, 4, D)
    row8 = lax.broadcasted_iota(jnp.int32, (D, 128), 0)
    col8 = lax.broadcasted_iota(jnp.int32, (D, 128), 1)
    acc = jnp.zeros((_WBM, 128), jnp.float32)
    for j in range(4):
        ej = (col8 == row8 + 32 * j).astype(jnp.float32)
        acc = acc + lax.dot_general(
            t3[:, j, :], ej, (((1,), (0,)), ((), ())),
            preferred_element_type=jnp.float32)
    o_ref[...] = acc


def _w_prep(w_t):
    out = pl.pallas_call(
        _wprep_body,
        grid=(pl.cdiv(VOCAB, 4 * _WBM),),
        in_specs=[pl.BlockSpec((D, 4 * _WBM), lambda i: (0, i))],
        out_specs=pl.BlockSpec((_WBM, 128), lambda i: (i, 0)),
        out_shape=jax.ShapeDtypeStruct((VOCAB * D // 128, 128), jnp.float32),
    )(w_t)
    return out.reshape(VOCAB, D)


def _sc_body(rows_hbm, cols_hbm, vals_hbm, w_hbm, out_hbm,
             rows_all, cols_all, vals_all, gbuf0, gbuf1, acc,
             sem_i, sem_g0, sem_g1, sem_s0, sem_s1):
    c = lax.axis_index("c")
    s = lax.axis_index("s")
    wid = c * NS + s
    seg0 = wid * SEGPW

    idx_descs = [
        pltpu.async_copy(rows_hbm.at[pl.ds(seg0, SEGPW)], rows_all, sem_i),
        pltpu.async_copy(cols_hbm.at[pl.ds(seg0, SEGPW)], cols_all, sem_i),
        pltpu.async_copy(vals_hbm.at[pl.ds(seg0, SEGPW)], vals_all, sem_i),
    ]

    zero = jnp.zeros((L,), jnp.float32)

    def zero_body(i, carry):
        gbuf0[i, pl.ds(0, L)] = zero
        gbuf0[i, pl.ds(L, L)] = zero
        return carry

    lax.fori_loop(0, CHUNK, zero_body, 0)
    pltpu.sync_copy(gbuf0, acc.at[pl.ds(s * RPS, RPS)])
    plsc.subcore_barrier()
    for dsc in idx_descs:
        dsc.wait()

    gbufs = (gbuf0, gbuf1)
    sems_g = (sem_g0, sem_g1)
    sems_s = (sem_s0, sem_s1)

    def fire_gather(k):
        gb, sem = gbufs[k % 2], sems_g[k % 2]
        return [
            pltpu.async_copy(w_hbm.at[cols_all.at[k * NSEG + j]],
                             gb.at[pl.ds(j * SEG, SEG)], sem)
            for j in range(NSEG)
        ]

    lane_ids = [jnp.full((L, 1), i, jnp.int32) for i in range(L)]
    gather_dn = lax.GatherDimensionNumbers(
        offset_dims=(), collapsed_slice_dims=(0,), start_index_map=(0,))

    descs_g = {0: fire_gather(0)}
    descs_s = {}
    for k in range(NCHUNK):
        gb = gbufs[k % 2]
        for dsc in descs_g.pop(k):
            dsc.wait()
        if k + 1 < NCHUNK:
            if k >= 1:
                for dsc in descs_s.pop(k - 1):
                    dsc.wait()
            descs_g[k + 1] = fire_gather(k + 1)

        def run_scale(gb, k):
            @plsc.parallel_loop(0, CHUNK // L, unroll=2)
            def scale_body(g):
                v = vals_all[k * NSEG + g // 8, pl.ds((g % 8) * L, L)]
                for i in range(L):
                    e = g * L + i
                    val = lax.gather(
                        v, lane_ids[i], gather_dn, slice_sizes=(1,),
                        mode=lax.GatherScatterMode.PROMISE_IN_BOUNDS)
                    gb[e, pl.ds(0, L)] = gb[e, pl.ds(0, L)] * val
                    gb[e, pl.ds(L, L)] = gb[e, pl.ds(L, L)] * val

        run_scale(gb, k)

        descs_s[k] = [
            pltpu.async_copy(gb.at[pl.ds(j * SEG, SEG)],
                             acc.at[rows_all.at[k * NSEG + j]],
                             sems_s[k % 2], add=True)
            for j in range(NSEG)
        ]
    for k in (NCHUNK - 2, NCHUNK - 1):
        for dsc in descs_s.pop(k):
            dsc.wait()
    plsc.subcore_barrier()

    pltpu.sync_copy(acc.at[pl.ds(s * RPS, RPS)],
                    out_hbm.at[c, pl.ds(s * RPS, RPS)])


_sc_accumulate = pl.kernel(
    _sc_body,
    out_type=jax.ShapeDtypeStruct((NC, BATCH, D), jnp.float32),
    mesh=plsc.VectorSubcoreMesh(core_axis_name="c", subcore_axis_name="s"),
    compiler_params=pltpu.CompilerParams(
        use_tc_tiling_on_sc=False, needs_layout_passes=False),
    scratch_types=[
        pltpu.VMEM((SEGPW, SEG), jnp.int32),
        pltpu.VMEM((SEGPW, SEG), jnp.int32),
        pltpu.VMEM((SEGPW, SEG), jnp.float32),
        pltpu.VMEM((CHUNK, D), jnp.float32),
        pltpu.VMEM((CHUNK, D), jnp.float32),
        pltpu.VMEM_SHARED((BATCH, D), jnp.float32),
        pltpu.SemaphoreType.DMA,
        pltpu.SemaphoreType.DMA,
        pltpu.SemaphoreType.DMA,
        pltpu.SemaphoreType.DMA,
        pltpu.SemaphoreType.DMA,
    ],
)


def _fin_body(p_hbm, d_hbm, b_hbm, out_hbm, p0b, p1b, dbuf, obuf, bbuf,
              sem_i, sem_o):
    c = lax.axis_index("c")
    s = lax.axis_index("s")
    wid = c * NS + s
    pltpu.sync_copy(b_hbm, bbuf)
    b0 = bbuf[pl.ds(0, L)]
    b1 = bbuf[pl.ds(L, L)]
    iota = lax.iota(jnp.int32, L)

    for it in range(RB // FB):
        r0 = wid * RB + it * FB
        in_descs = [
            pltpu.async_copy(p_hbm.at[0, pl.ds(r0, FB)], p0b, sem_i),
            pltpu.async_copy(p_hbm.at[1, pl.ds(r0, FB)], p1b, sem_i),
            pltpu.async_copy(d_hbm.at[pl.ds(r0, FB)], dbuf, sem_i),
        ]
        for dsc in in_descs:
            dsc.wait()

        @plsc.parallel_loop(0, FB, unroll=2)
        def ent_body(r):
            rv = jnp.full((L,), r, jnp.int32)
            sp0 = p0b[r, pl.ds(0, L)] + p1b[r, pl.ds(0, L)] + b0
            sp1 = p0b[r, pl.ds(L, L)] + p1b[r, pl.ds(L, L)] + b1
            plsc.store_scatter(obuf, [iota, rv], sp0)
            plsc.store_scatter(obuf, [iota + L, rv], sp1)
            for h in range(DU // L):
                dv = dbuf[r, pl.ds(h * L, L)]
                plsc.store_scatter(obuf, [iota + (D + h * L), rv], dv)

        pltpu.async_copy(obuf.at[:, pl.ds(0, FB)],
                         out_hbm.at[:, pl.ds(r0, FB)], sem_o).wait()


_sc_finish = pl.kernel(
    _fin_body,
    out_type=jax.ShapeDtypeStruct((NOUT, BATCH), jnp.float32),
    mesh=plsc.VectorSubcoreMesh(core_axis_name="c", subcore_axis_name="s"),
    compiler_params=pltpu.CompilerParams(
        use_tc_tiling_on_sc=False, needs_layout_passes=False),
    scratch_types=[
        pltpu.VMEM((FB, D), jnp.float32),
        pltpu.VMEM((FB, D), jnp.float32),
        pltpu.VMEM((FB, DU), jnp.float32),
        pltpu.VMEM((NOUT, FBP), jnp.float32),
        pltpu.VMEM((D,), jnp.float32),
        pltpu.SemaphoreType.DMA,
        pltpu.SemaphoreType.DMA,
    ],
)


def kernel(sparse_rows, sparse_cols, sparse_vals, dense_feat, W, b):
    rows2d = sparse_rows.astype(jnp.int32).reshape(NNZ // SEG, SEG)
    cols2d = sparse_cols.astype(jnp.int32).reshape(NNZ // SEG, SEG)
    vals2d = sparse_vals.reshape(NNZ // SEG, SEG)
    w_sc = _w_prep(W.T)
    partial = _sc_accumulate(rows2d, cols2d, vals2d, w_sc)
    out_t = _sc_finish(partial, dense_feat, b)
    return out_t.T


# speedup vs baseline: 1.0118x; 1.0034x over previous
import functools

import jax
import jax.numpy as jnp
import numpy as np
from jax import lax
from jax.experimental import pallas as pl
from jax.experimental.pallas import tpu as pltpu
from jax.experimental.pallas import tpu_sc as plsc

BATCH = 16384
VOCAB = 100000
NNZ = 327680
D = 32
DU = 128
NOUT = D + DU

NC = 2
NS = 16
NW = NC * NS
L = 16

CHUNK = 640
SEG = 128
NSEG = CHUNK // SEG
EPW = NNZ // NW
NCHUNK = EPW // CHUNK
SEGPW = EPW // SEG
RPS = BATCH // NS

RB = BATCH // NW
FB = 256
FBP = 264


_WBM = 1024


def _wprep_body(wt_ref, o_ref):
    row = lax.broadcasted_iota(jnp.int32, (D, D), 0)
    col = lax.broadcasted_iota(jnp.int32, (D, D), 1)
    eye = (row == col).astype(jnp.float32)
    t = lax.dot_general(wt_ref[...], eye, (((0,), (0,)), ((), ())),
                        preferred_element_type=jnp.float32)
    t3 = t.reshape(_WBM, 4, D)
    row8 = lax.broadcasted_iota(jnp.int32, (D, 128), 0)
    col8 = lax.broadcasted_iota(jnp.int32, (D, 128), 1)
    acc = jnp.zeros((_WBM, 128), jnp.float32)
    for j in range(4):
        ej = (col8 == row8 + 32 * j).astype(jnp.float32)
        acc = acc + lax.dot_general(
            t3[:, j, :], ej, (((1,), (0,)), ((), ())),
            preferred_element_type=jnp.float32)
    o_ref[...] = acc


def _w_prep(w_t):
    out = pl.pallas_call(
        _wprep_body,
        grid=(pl.cdiv(VOCAB, 4 * _WBM),),
        in_specs=[pl.BlockSpec((D, 4 * _WBM), lambda i: (0, i))],
        out_specs=pl.BlockSpec((_WBM, 128), lambda i: (i, 0)),
        out_shape=jax.ShapeDtypeStruct((VOCAB * D // 128, 128), jnp.float32),
    )(w_t)
    return out.reshape(VOCAB, D)


def _sc_body(rows_hbm, cols_hbm, vals_hbm, w_hbm, out_hbm,
             rows_all, cols_all, vals_all, gbuf0, gbuf1, gbuf2, acc,
             sem_i, sem_g0, sem_g1, sem_g2, sem_s0, sem_s1, sem_s2):
    c = lax.axis_index("c")
    s = lax.axis_index("s")
    wid = c * NS + s
    seg0 = wid * SEGPW

    idx_descs = [
        pltpu.async_copy(rows_hbm.at[pl.ds(seg0, SEGPW)], rows_all, sem_i),
        pltpu.async_copy(cols_hbm.at[pl.ds(seg0, SEGPW)], cols_all, sem_i),
        pltpu.async_copy(vals_hbm.at[pl.ds(seg0, SEGPW)], vals_all, sem_i),
    ]

    zero = jnp.zeros((L,), jnp.float32)

    def zero_body(i, carry):
        gbuf0[i, pl.ds(0, L)] = zero
        gbuf0[i, pl.ds(L, L)] = zero
        return carry

    lax.fori_loop(0, CHUNK, zero_body, 0)
    for h in range(RPS // 512):
        pltpu.sync_copy(gbuf0.at[pl.ds(0, 512)],
                        acc.at[pl.ds(s * RPS + h * 512, 512)])
    plsc.subcore_barrier()
    for dsc in idx_descs:
        dsc.wait()

    gbufs = (gbuf0, gbuf1, gbuf2)
    sems_g = (sem_g0, sem_g1, sem_g2)
    sems_s = (sem_s0, sem_s1, sem_s2)

    def fire_gather(k):
        gb, sem = gbufs[k % 3], sems_g[k % 3]
        return [
            pltpu.async_copy(w_hbm.at[cols_all.at[k * NSEG + j]],
                             gb.at[pl.ds(j * SEG, SEG)], sem)
            for j in range(NSEG)
        ]

    lane_ids = [jnp.full((L, 1), i, jnp.int32) for i in range(L)]
    gather_dn = lax.GatherDimensionNumbers(
        offset_dims=(), collapsed_slice_dims=(0,), start_index_map=(0,))

    descs_g = {0: fire_gather(0)}
    descs_s = {}
    for k in range(NCHUNK):
        gb = gbufs[k % 3]
        for dsc in descs_g.pop(k):
            dsc.wait()
        if k + 1 < NCHUNK:
            if k >= 2:
                for dsc in descs_s.pop(k - 2):
                    dsc.wait()
            descs_g[k + 1] = fire_gather(k + 1)

        def run_scale(gb, k):
            @plsc.parallel_loop(0, CHUNK // L, unroll=2)
            def scale_body(g):
                v = vals_all[k * NSEG + g // 8, pl.ds((g % 8) * L, L)]
                for i in range(L):
                    e = g * L + i
                    val = lax.gather(
                        v, lane_ids[i], gather_dn, slice_sizes=(1,),
                        mode=lax.GatherScatterMode.PROMISE_IN_BOUNDS)
                    gb[e, pl.ds(0, L)] = gb[e, pl.ds(0, L)] * val
                    gb[e, pl.ds(L, L)] = gb[e, pl.ds(L, L)] * val

        run_scale(gb, k)

        descs_s[k] = [
            pltpu.async_copy(gb.at[pl.ds(j * SEG, SEG)],
                             acc.at[rows_all.at[k * NSEG + j]],
                             sems_s[k % 3], add=True)
            for j in range(NSEG)
        ]
    for k in (NCHUNK - 3, NCHUNK - 2, NCHUNK - 1):
        for dsc in descs_s.pop(k):
            dsc.wait()
    plsc.subcore_barrier()

    pltpu.sync_copy(acc.at[pl.ds(s * RPS, RPS)],
                    out_hbm.at[c, pl.ds(s * RPS, RPS)])


_sc_accumulate = pl.kernel(
    _sc_body,
    out_type=jax.ShapeDtypeStruct((NC, BATCH, D), jnp.float32),
    mesh=plsc.VectorSubcoreMesh(core_axis_name="c", subcore_axis_name="s"),
    compiler_params=pltpu.CompilerParams(
        use_tc_tiling_on_sc=False, needs_layout_passes=False),
    scratch_types=[
        pltpu.VMEM((SEGPW, SEG), jnp.int32),
        pltpu.VMEM((SEGPW, SEG), jnp.int32),
        pltpu.VMEM((SEGPW, SEG), jnp.float32),
        pltpu.VMEM((CHUNK, D), jnp.float32),
        pltpu.VMEM((CHUNK, D), jnp.float32),
        pltpu.VMEM((CHUNK, D), jnp.float32),
        pltpu.VMEM_SHARED((BATCH, D), jnp.float32),
        pltpu.SemaphoreType.DMA,
        pltpu.SemaphoreType.DMA,
        pltpu.SemaphoreType.DMA,
        pltpu.SemaphoreType.DMA,
        pltpu.SemaphoreType.DMA,
        pltpu.SemaphoreType.DMA,
        pltpu.SemaphoreType.DMA,
    ],
)


def _fin_body(p_hbm, d_hbm, b_hbm, out_hbm, p0b, p1b, dbuf, obuf, bbuf,
              sem_i, sem_o):
    c = lax.axis_index("c")
    s = lax.axis_index("s")
    wid = c * NS + s
    pltpu.sync_copy(b_hbm, bbuf)
    b0 = bbuf[pl.ds(0, L)]
    b1 = bbuf[pl.ds(L, L)]
    iota = lax.iota(jnp.int32, L)

    for it in range(RB // FB):
        r0 = wid * RB + it * FB
        in_descs = [
            pltpu.async_copy(p_hbm.at[0, pl.ds(r0, FB)], p0b, sem_i),
            pltpu.async_copy(p_hbm.at[1, pl.ds(r0, FB)], p1b, sem_i),
            pltpu.async_copy(d_hbm.at[pl.ds(r0, FB)], dbuf, sem_i),
        ]
        for dsc in in_descs:
            dsc.wait()

        @plsc.parallel_loop(0, FB, unroll=2)
        def ent_body(r):
            rv = jnp.full((L,), r, jnp.int32)
            sp0 = p0b[r, pl.ds(0, L)] + p1b[r, pl.ds(0, L)] + b0
            sp1 = p0b[r, pl.ds(L, L)] + p1b[r, pl.ds(L, L)] + b1
            plsc.store_scatter(obuf, [iota, rv], sp0)
            plsc.store_scatter(obuf, [iota + L, rv], sp1)
            for h in range(DU // L):
                dv = dbuf[r, pl.ds(h * L, L)]
                plsc.store_scatter(obuf, [iota + (D + h * L), rv], dv)

        pltpu.async_copy(obuf.at[:, pl.ds(0, FB)],
                         out_hbm.at[:, pl.ds(r0, FB)], sem_o).wait()


_sc_finish = pl.kernel(
    _fin_body,
    out_type=jax.ShapeDtypeStruct((NOUT, BATCH), jnp.float32),
    mesh=plsc.VectorSubcoreMesh(core_axis_name="c", subcore_axis_name="s"),
    compiler_params=pltpu.CompilerParams(
        use_tc_tiling_on_sc=False, needs_layout_passes=False),
    scratch_types=[
        pltpu.VMEM((FB, D), jnp.float32),
        pltpu.VMEM((FB, D), jnp.float32),
        pltpu.VMEM((FB, DU), jnp.float32),
        pltpu.VMEM((NOUT, FBP), jnp.float32),
        pltpu.VMEM((D,), jnp.float32),
        pltpu.SemaphoreType.DMA,
        pltpu.SemaphoreType.DMA,
    ],
)


def kernel(sparse_rows, sparse_cols, sparse_vals, dense_feat, W, b):
    rows2d = sparse_rows.astype(jnp.int32).reshape(NNZ // SEG, SEG)
    cols2d = sparse_cols.astype(jnp.int32).reshape(NNZ // SEG, SEG)
    vals2d = sparse_vals.reshape(NNZ // SEG, SEG)
    w_sc = _w_prep(W.T)
    partial = _sc_accumulate(rows2d, cols2d, vals2d, w_sc)
    out_t = _sc_finish(partial, dense_feat, b)
    return out_t.T


# speedup vs baseline: 1.0120x; 1.0001x over previous
import functools

import jax
import jax.numpy as jnp
import numpy as np
from jax import lax
from jax.experimental import pallas as pl
from jax.experimental.pallas import tpu as pltpu
from jax.experimental.pallas import tpu_sc as plsc

BATCH = 16384
VOCAB = 100000
NNZ = 327680
D = 32
DU = 128
NOUT = D + DU

NC = 2
NS = 16
NW = NC * NS
L = 16

CHUNK = 640
SEG = 128
NSEG = CHUNK // SEG
EPW = NNZ // NW
NCHUNK = EPW // CHUNK
SEGPW = EPW // SEG
RPS = BATCH // NS

RB = BATCH // NW
FB = 256
FBP = 264


_WBM = 1024


def _wprep_body(wt_ref, o_ref):
    row = lax.broadcasted_iota(jnp.int32, (D, D), 0)
    col = lax.broadcasted_iota(jnp.int32, (D, D), 1)
    eye = (row == col).astype(jnp.float32)
    t = lax.dot_general(wt_ref[...], eye, (((0,), (0,)), ((), ())),
                        preferred_element_type=jnp.float32)
    t3 = t.reshape(_WBM, 4, D)
    row8 = lax.broadcasted_iota(jnp.int32, (D, 128), 0)
    col8 = lax.broadcasted_iota(jnp.int32, (D, 128), 1)
    acc = jnp.zeros((_WBM, 128), jnp.float32)
    for j in range(4):
        ej = (col8 == row8 + 32 * j).astype(jnp.float32)
        acc = acc + lax.dot_general(
            t3[:, j, :], ej, (((1,), (0,)), ((), ())),
            preferred_element_type=jnp.float32)
    o_ref[...] = acc


def _w_prep(w_t):
    out = pl.pallas_call(
        _wprep_body,
        grid=(pl.cdiv(VOCAB, 4 * _WBM),),
        in_specs=[pl.BlockSpec((D, 4 * _WBM), lambda i: (0, i))],
        out_specs=pl.BlockSpec((_WBM, 128), lambda i: (i, 0)),
        out_shape=jax.ShapeDtypeStruct((VOCAB * D // 128, 128), jnp.float32),
    )(w_t)
    return out.reshape(VOCAB, D)


def _sc_body(rows_hbm, cols_hbm, vals_hbm, w_hbm, out_hbm,
             rows_all, cols_all, vals_all, gbuf0, gbuf1, gbuf2, acc,
             sem_i, sem_g0, sem_g1, sem_g2, sem_s0, sem_s1, sem_s2):
    c = lax.axis_index("c")
    s = lax.axis_index("s")
    wid = c * NS + s
    seg0 = wid * SEGPW

    idx_descs = [
        pltpu.async_copy(rows_hbm.at[pl.ds(seg0, SEGPW)], rows_all, sem_i),
        pltpu.async_copy(cols_hbm.at[pl.ds(seg0, SEGPW)], cols_all, sem_i),
        pltpu.async_copy(vals_hbm.at[pl.ds(seg0, SEGPW)], vals_all, sem_i),
    ]

    zero = jnp.zeros((L,), jnp.float32)

    def zero_body(i, carry):
        gbuf0[i, pl.ds(0, L)] = zero
        gbuf0[i, pl.ds(L, L)] = zero
        return carry

    lax.fori_loop(0, CHUNK, zero_body, 0)
    for h in range(RPS // 512):
        pltpu.sync_copy(gbuf0.at[pl.ds(0, 512)],
                        acc.at[pl.ds(s * RPS + h * 512, 512)])
    plsc.subcore_barrier()
    for dsc in idx_descs:
        dsc.wait()

    gbufs = (gbuf0, gbuf1, gbuf2)
    sems_g = (sem_g0, sem_g1, sem_g2)
    sems_s = (sem_s0, sem_s1, sem_s2)

    def fire_gather(k):
        gb, sem = gbufs[k % 3], sems_g[k % 3]
        return [
            pltpu.async_copy(w_hbm.at[cols_all.at[k * NSEG + j]],
                             gb.at[pl.ds(j * SEG, SEG)], sem)
            for j in range(NSEG)
        ]

    lane_ids = [jnp.full((L, 1), i, jnp.int32) for i in range(L)]
    gather_dn = lax.GatherDimensionNumbers(
        offset_dims=(), collapsed_slice_dims=(0,), start_index_map=(0,))

    descs_g = {0: fire_gather(0)}
    descs_s = {}
    for k in range(NCHUNK):
        gb = gbufs[k % 3]
        if k + 1 < NCHUNK:
            if k >= 2:
                for dsc in descs_s.pop(k - 2):
                    dsc.wait()
            descs_g[k + 1] = fire_gather(k + 1)
        for dsc in descs_g.pop(k):
            dsc.wait()

        def run_scale(gb, k):
            @plsc.parallel_loop(0, CHUNK // L, unroll=2)
            def scale_body(g):
                v = vals_all[k * NSEG + g // 8, pl.ds((g % 8) * L, L)]
                for i in range(L):
                    e = g * L + i
                    val = lax.gather(
                        v, lane_ids[i], gather_dn, slice_sizes=(1,),
                        mode=lax.GatherScatterMode.PROMISE_IN_BOUNDS)
                    gb[e, pl.ds(0, L)] = gb[e, pl.ds(0, L)] * val
                    gb[e, pl.ds(L, L)] = gb[e, pl.ds(L, L)] * val

        run_scale(gb, k)

        descs_s[k] = [
            pltpu.async_copy(gb.at[pl.ds(j * SEG, SEG)],
                             acc.at[rows_all.at[k * NSEG + j]],
                             sems_s[k % 3], add=True)
            for j in range(NSEG)
        ]
    for k in (NCHUNK - 3, NCHUNK - 2, NCHUNK - 1):
        for dsc in descs_s.pop(k):
            dsc.wait()
    plsc.subcore_barrier()

    pltpu.sync_copy(acc.at[pl.ds(s * RPS, RPS)],
                    out_hbm.at[c, pl.ds(s * RPS, RPS)])


_sc_accumulate = pl.kernel(
    _sc_body,
    out_type=jax.ShapeDtypeStruct((NC, BATCH, D), jnp.float32),
    mesh=plsc.VectorSubcoreMesh(core_axis_name="c", subcore_axis_name="s"),
    compiler_params=pltpu.CompilerParams(
        use_tc_tiling_on_sc=False, needs_layout_passes=False),
    scratch_types=[
        pltpu.VMEM((SEGPW, SEG), jnp.int32),
        pltpu.VMEM((SEGPW, SEG), jnp.int32),
        pltpu.VMEM((SEGPW, SEG), jnp.float32),
        pltpu.VMEM((CHUNK, D), jnp.float32),
        pltpu.VMEM((CHUNK, D), jnp.float32),
        pltpu.VMEM((CHUNK, D), jnp.float32),
        pltpu.VMEM_SHARED((BATCH, D), jnp.float32),
        pltpu.SemaphoreType.DMA,
        pltpu.SemaphoreType.DMA,
        pltpu.SemaphoreType.DMA,
        pltpu.SemaphoreType.DMA,
        pltpu.SemaphoreType.DMA,
        pltpu.SemaphoreType.DMA,
        pltpu.SemaphoreType.DMA,
    ],
)


def _fin_body(p_hbm, d_hbm, b_hbm, out_hbm, p0b, p1b, dbuf, obuf, bbuf,
              sem_i, sem_o):
    c = lax.axis_index("c")
    s = lax.axis_index("s")
    wid = c * NS + s
    pltpu.sync_copy(b_hbm, bbuf)
    b0 = bbuf[pl.ds(0, L)]
    b1 = bbuf[pl.ds(L, L)]
    iota = lax.iota(jnp.int32, L)

    for it in range(RB // FB):
        r0 = wid * RB + it * FB
        in_descs = [
            pltpu.async_copy(p_hbm.at[0, pl.ds(r0, FB)], p0b, sem_i),
            pltpu.async_copy(p_hbm.at[1, pl.ds(r0, FB)], p1b, sem_i),
            pltpu.async_copy(d_hbm.at[pl.ds(r0, FB)], dbuf, sem_i),
        ]
        for dsc in in_descs:
            dsc.wait()

        @plsc.parallel_loop(0, FB, unroll=2)
        def ent_body(r):
            rv = jnp.full((L,), r, jnp.int32)
            sp0 = p0b[r, pl.ds(0, L)] + p1b[r, pl.ds(0, L)] + b0
            sp1 = p0b[r, pl.ds(L, L)] + p1b[r, pl.ds(L, L)] + b1
            plsc.store_scatter(obuf, [iota, rv], sp0)
            plsc.store_scatter(obuf, [iota + L, rv], sp1)
            for h in range(DU // L):
                dv = dbuf[r, pl.ds(h * L, L)]
                plsc.store_scatter(obuf, [iota + (D + h * L), rv], dv)

        pltpu.async_copy(obuf.at[:, pl.ds(0, FB)],
                         out_hbm.at[:, pl.ds(r0, FB)], sem_o).wait()


_sc_finish = pl.kernel(
    _fin_body,
    out_type=jax.ShapeDtypeStruct((NOUT, BATCH), jnp.float32),
    mesh=plsc.VectorSubcoreMesh(core_axis_name="c", subcore_axis_name="s"),
    compiler_params=pltpu.CompilerParams(
        use_tc_tiling_on_sc=False, needs_layout_passes=False),
    scratch_types=[
        pltpu.VMEM((FB, D), jnp.float32),
        pltpu.VMEM((FB, D), jnp.float32),
        pltpu.VMEM((FB, DU), jnp.float32),
        pltpu.VMEM((NOUT, FBP), jnp.float32),
        pltpu.VMEM((D,), jnp.float32),
        pltpu.SemaphoreType.DMA,
        pltpu.SemaphoreType.DMA,
    ],
)


def kernel(sparse_rows, sparse_cols, sparse_vals, dense_feat, W, b):
    rows2d = sparse_rows.astype(jnp.int32).reshape(NNZ // SEG, SEG)
    cols2d = sparse_cols.astype(jnp.int32).reshape(NNZ // SEG, SEG)
    vals2d = sparse_vals.reshape(NNZ // SEG, SEG)
    w_sc = _w_prep(W.T)
    partial = _sc_accumulate(rows2d, cols2d, vals2d, w_sc)
    out_t = _sc_finish(partial, dense_feat, b)
    return out_t.T


# speedup vs baseline: 1.0137x; 1.0017x over previous
import jax
import jax.numpy as jnp
from jax import lax
from jax.experimental import pallas as pl
from jax.experimental.pallas import tpu as pltpu
from jax.experimental.pallas import tpu_sc as plsc

BATCH = 16384
VOCAB = 100000
NNZ = 327680
D = 32
DU = 128
NOUT = D + DU

NC = 2
NS = 16
NW = NC * NS
L = 16

CHUNK = 640
SEG = 128
NSEG = CHUNK // SEG
EPW = NNZ // NW
NCHUNK = EPW // CHUNK
SEGPW = EPW // SEG
RPS = BATCH // NS

RB = BATCH // NW
FB = 256
FBP = 264


_WBM = 1024


def _wprep_body(wt_ref, o_ref):
    row = lax.broadcasted_iota(jnp.int32, (D, D), 0)
    col = lax.broadcasted_iota(jnp.int32, (D, D), 1)
    eye = (row == col).astype(jnp.float32)
    t = lax.dot_general(wt_ref[...], eye, (((0,), (0,)), ((), ())),
                        preferred_element_type=jnp.float32)
    t3 = t.reshape(_WBM, 4, D)
    row8 = lax.broadcasted_iota(jnp.int32, (D, 128), 0)
    col8 = lax.broadcasted_iota(jnp.int32, (D, 128), 1)
    acc = jnp.zeros((_WBM, 128), jnp.float32)
    for j in range(4):
        ej = (col8 == row8 + 32 * j).astype(jnp.float32)
        acc = acc + lax.dot_general(
            t3[:, j, :], ej, (((1,), (0,)), ((), ())),
            preferred_element_type=jnp.float32)
    o_ref[...] = acc


def _w_prep(w_t):
    out = pl.pallas_call(
        _wprep_body,
        grid=(pl.cdiv(VOCAB, 4 * _WBM),),
        in_specs=[pl.BlockSpec((D, 4 * _WBM), lambda i: (0, i))],
        out_specs=pl.BlockSpec((_WBM, 128), lambda i: (i, 0)),
        out_shape=jax.ShapeDtypeStruct((VOCAB * D // 128, 128), jnp.float32),
    )(w_t)
    return out.reshape(VOCAB, D)


def _sc_body(rows_hbm, cols_hbm, vals_hbm, w_hbm, out_hbm,
             rows_all, cols_all, vals_all, gbuf0, gbuf1, gbuf2, acc,
             sem_i, sem_g0, sem_g1, sem_g2, sem_s0, sem_s1, sem_s2):
    c = lax.axis_index("c")
    s = lax.axis_index("s")
    wid = c * NS + s
    seg0 = wid * SEGPW

    idx_descs = [
        pltpu.async_copy(rows_hbm.at[pl.ds(seg0, SEGPW)], rows_all, sem_i),
        pltpu.async_copy(cols_hbm.at[pl.ds(seg0, SEGPW)], cols_all, sem_i),
        pltpu.async_copy(vals_hbm.at[pl.ds(seg0, SEGPW)], vals_all, sem_i),
    ]

    zero = jnp.zeros((L,), jnp.float32)

    def zero_body(i, carry):
        gbuf0[i, pl.ds(0, L)] = zero
        gbuf0[i, pl.ds(L, L)] = zero
        return carry

    lax.fori_loop(0, CHUNK, zero_body, 0)
    for h in range(RPS // 512):
        pltpu.sync_copy(gbuf0.at[pl.ds(0, 512)],
                        acc.at[pl.ds(s * RPS + h * 512, 512)])
    plsc.subcore_barrier()
    for dsc in idx_descs:
        dsc.wait()

    gbufs = (gbuf0, gbuf1, gbuf2)
    sems_g = (sem_g0, sem_g1, sem_g2)
    sems_s = (sem_s0, sem_s1, sem_s2)

    def fire_gather(k):
        gb, sem = gbufs[k % 3], sems_g[k % 3]
        return [
            pltpu.async_copy(w_hbm.at[cols_all.at[k * NSEG + j]],
                             gb.at[pl.ds(j * SEG, SEG)], sem)
            for j in range(NSEG)
        ]

    lane_ids = [jnp.full((L, 1), i, jnp.int32) for i in range(L)]
    gather_dn = lax.GatherDimensionNumbers(
        offset_dims=(), collapsed_slice_dims=(0,), start_index_map=(0,))

    descs_g = {0: fire_gather(0)}
    descs_s = {}
    for k in range(NCHUNK):
        gb = gbufs[k % 3]
        if k + 1 < NCHUNK:
            if k >= 2:
                for dsc in descs_s.pop(k - 2):
                    dsc.wait()
            descs_g[k + 1] = fire_gather(k + 1)
        for dsc in descs_g.pop(k):
            dsc.wait()

        def run_scale(gb, k):
            @plsc.parallel_loop(0, CHUNK // L, unroll=2)
            def scale_body(g):
                v = vals_all[k * NSEG + g // 8, pl.ds((g % 8) * L, L)]
                for i in range(L):
                    e = g * L + i
                    val = lax.gather(
                        v, lane_ids[i], gather_dn, slice_sizes=(1,),
                        mode=lax.GatherScatterMode.PROMISE_IN_BOUNDS)
                    gb[e, pl.ds(0, L)] = gb[e, pl.ds(0, L)] * val
                    gb[e, pl.ds(L, L)] = gb[e, pl.ds(L, L)] * val

        run_scale(gb, k)

        descs_s[k] = [
            pltpu.async_copy(gb.at[pl.ds(j * SEG, SEG)],
                             acc.at[rows_all.at[k * NSEG + j]],
                             sems_s[k % 3], add=True)
            for j in range(NSEG)
        ]
    for k in (NCHUNK - 3, NCHUNK - 2, NCHUNK - 1):
        for dsc in descs_s.pop(k):
            dsc.wait()
    plsc.subcore_barrier()

    pltpu.sync_copy(acc.at[pl.ds(s * RPS, RPS)],
                    out_hbm.at[c, pl.ds(s * RPS, RPS)])


_sc_accumulate = pl.kernel(
    _sc_body,
    out_type=jax.ShapeDtypeStruct((NC, BATCH, D), jnp.float32),
    mesh=plsc.VectorSubcoreMesh(core_axis_name="c", subcore_axis_name="s"),
    compiler_params=pltpu.CompilerParams(
        use_tc_tiling_on_sc=False, needs_layout_passes=False),
    scratch_types=[
        pltpu.VMEM((SEGPW, SEG), jnp.int32),
        pltpu.VMEM((SEGPW, SEG), jnp.int32),
        pltpu.VMEM((SEGPW, SEG), jnp.float32),
        pltpu.VMEM((CHUNK, D), jnp.float32),
        pltpu.VMEM((CHUNK, D), jnp.float32),
        pltpu.VMEM((CHUNK, D), jnp.float32),
        pltpu.VMEM_SHARED((BATCH, D), jnp.float32),
        pltpu.SemaphoreType.DMA,
        pltpu.SemaphoreType.DMA,
        pltpu.SemaphoreType.DMA,
        pltpu.SemaphoreType.DMA,
        pltpu.SemaphoreType.DMA,
        pltpu.SemaphoreType.DMA,
        pltpu.SemaphoreType.DMA,
    ],
)


def _fin_body(p_hbm, d_hbm, b_hbm, out_hbm, p0b, p1b, dbuf, obuf, bbuf,
              sem_i, sem_o):
    c = lax.axis_index("c")
    s = lax.axis_index("s")
    wid = c * NS + s
    pltpu.sync_copy(b_hbm, bbuf)
    b0 = bbuf[pl.ds(0, L)]
    b1 = bbuf[pl.ds(L, L)]
    iota = lax.iota(jnp.int32, L)

    for it in range(RB // FB):
        r0 = wid * RB + it * FB
        in_descs = [
            pltpu.async_copy(p_hbm.at[0, pl.ds(r0, FB)], p0b, sem_i),
            pltpu.async_copy(p_hbm.at[1, pl.ds(r0, FB)], p1b, sem_i),
            pltpu.async_copy(d_hbm.at[pl.ds(r0, FB)], dbuf, sem_i),
        ]
        for dsc in in_descs:
            dsc.wait()

        @plsc.parallel_loop(0, FB, unroll=2)
        def ent_body(r):
            rv = jnp.full((L,), r, jnp.int32)
            sp0 = p0b[r, pl.ds(0, L)] + p1b[r, pl.ds(0, L)] + b0
            sp1 = p0b[r, pl.ds(L, L)] + p1b[r, pl.ds(L, L)] + b1
            plsc.store_scatter(obuf, [iota, rv], sp0)
            plsc.store_scatter(obuf, [iota + L, rv], sp1)
            for h in range(DU // L):
                dv = dbuf[r, pl.ds(h * L, L)]
                plsc.store_scatter(obuf, [iota + (D + h * L), rv], dv)

        pltpu.async_copy(obuf.at[:, pl.ds(0, FB)],
                         out_hbm.at[:, pl.ds(r0, FB)], sem_o).wait()


_sc_finish = pl.kernel(
    _fin_body,
    out_type=jax.ShapeDtypeStruct((NOUT, BATCH), jnp.float32),
    mesh=plsc.VectorSubcoreMesh(core_axis_name="c", subcore_axis_name="s"),
    compiler_params=pltpu.CompilerParams(
        use_tc_tiling_on_sc=False, needs_layout_passes=False),
    scratch_types=[
        pltpu.VMEM((FB, D), jnp.float32),
        pltpu.VMEM((FB, D), jnp.float32),
        pltpu.VMEM((FB, DU), jnp.float32),
        pltpu.VMEM((NOUT, FBP), jnp.float32),
        pltpu.VMEM((D,), jnp.float32),
        pltpu.SemaphoreType.DMA,
        pltpu.SemaphoreType.DMA,
    ],
)


def kernel(sparse_rows, sparse_cols, sparse_vals, dense_feat, W, b):
    rows2d = sparse_rows.astype(jnp.int32).reshape(NNZ // SEG, SEG)
    cols2d = sparse_cols.astype(jnp.int32).reshape(NNZ // SEG, SEG)
    vals2d = sparse_vals.reshape(NNZ // SEG, SEG)
    w_sc = _w_prep(W.T)
    partial = _sc_accumulate(rows2d, cols2d, vals2d, w_sc)
    out_t = _sc_finish(partial, dense_feat, b)
    return out_t.T
